# Initial kernel scaffold; baseline (speedup 1.0000x reference)
import jax
import jax.numpy as jnp
from jax.experimental import pallas as pl


def kernel(x1, x2, edge_index):
    raise NotImplementedError("write your pallas kernel here")


# speedup vs baseline: 2.5876x; 2.5876x over previous
import functools

import jax
import jax.numpy as jnp
from jax import lax
from jax.experimental import pallas as pl
from jax.experimental.pallas import tpu as pltpu
from jax.experimental.pallas import tpu_sc as plsc

N_NODES = 10000
E = 320000
D = 128

NC = 2
NS = 16
NW = NC * NS

CHUNK = 128
EDGES_PER_W = -(-E // (NW * CHUNK)) * CHUNK
NCHUNK = EDGES_PER_W // CHUNK
E_PAD = EDGES_PER_W * NW


def _sc_kernel_body(x_hbm, src_hbm, dst_hbm, out_hbm,
                    idx_src_v, idx_dst_v, src_rows, dst_rows, out_v, tr,
                    sem_s, sem_d):
    wid = lax.axis_index("s") * NC + lax.axis_index("c")

    pltpu.sync_copy(src_hbm.at[wid], idx_src_v)
    pltpu.sync_copy(dst_hbm.at[wid], idx_dst_v)

    def chunk_body(c, _):
        cp_s = pltpu.async_copy(x_hbm.at[idx_src_v.at[c]], src_rows, sem_s)
        cp_d = pltpu.async_copy(x_hbm.at[idx_dst_v.at[c]], dst_rows, sem_d)
        cp_s.wait()
        cp_d.wait()

        lane16 = lax.iota(jnp.int32, 16) * 16

        def group_body(g, _):
            for el in range(16):
                e = g * 16 + el
                acc = jnp.zeros((16,), jnp.float32)
                for j in range(D // 16):
                    s = src_rows[e, pl.ds(j * 16, 16)]
                    d = dst_rows[e, pl.ds(j * 16, 16)]
                    acc = acc + s * d
                tr[pl.ds(el * 16, 16)] = acc

            tot = jnp.zeros((16,), jnp.float32)
            for f in range(16):
                tot = tot + plsc.load_gather(tr, [lane16 + f])
            out_v[c, pl.ds(g * 16, 16)] = tot
            return 0

        lax.fori_loop(0, CHUNK // 16, group_body, 0)
        return 0

    lax.fori_loop(0, NCHUNK, chunk_body, 0)

    pltpu.sync_copy(out_v, out_hbm.at[wid])


@jax.jit
def kernel(x1, x2, edge_index):
    x = jnp.concatenate([x1, x2], axis=0)
    pad = E_PAD - E
    src = jnp.pad(edge_index[0], (0, pad)).reshape(NW, NCHUNK, CHUNK)
    dst = jnp.pad(edge_index[1], (0, pad)).reshape(NW, NCHUNK, CHUNK)

    mesh = plsc.VectorSubcoreMesh(core_axis_name="c", subcore_axis_name="s")
    run = pl.kernel(
        _sc_kernel_body,
        out_type=jax.ShapeDtypeStruct((NW, NCHUNK, CHUNK), jnp.float32),
        mesh=mesh,
        compiler_params=pltpu.CompilerParams(needs_layout_passes=False),
        scratch_types=[
            pltpu.VMEM((NCHUNK, CHUNK), jnp.int32),
            pltpu.VMEM((NCHUNK, CHUNK), jnp.int32),
            pltpu.VMEM((CHUNK, D), jnp.float32),
            pltpu.VMEM((CHUNK, D), jnp.float32),
            pltpu.VMEM((NCHUNK, CHUNK), jnp.float32),
            pltpu.VMEM((256,), jnp.float32),
            pltpu.SemaphoreType.DMA,
            pltpu.SemaphoreType.DMA,
        ],
    )
    out = run(x, src, dst)
    return out.reshape(E_PAD)[:E]
